# baseline (speedup 1.0000x reference)
import functools

import jax
import jax.numpy as jnp
from jax import lax
from jax.experimental import pallas as pl
from jax.experimental.pallas import tpu as pltpu
from jax.experimental.pallas import tpu_sc as plsc

NC = 2
NS = 16
NW = NC * NS
CH = 512


def _proj_body(x_ref, w0t_ref, w1t_ref, b0_ref, b1_ref, o0_ref, o1_ref):
    x = x_ref[...]
    o0_ref[...] = jnp.dot(x, w0t_ref[...], preferred_element_type=jnp.float32) + b0_ref[...]
    o1_ref[...] = jnp.dot(x, w1t_ref[...], preferred_element_type=jnp.float32) + b1_ref[...]


def _combine_body(vw0_ref, p0_ref, p1_ref, o_ref):
    s = p0_ref[...] + p1_ref[...]
    o_ref[...] = vw0_ref[...] + jnp.concatenate([s, jnp.zeros_like(s)], axis=1)


def _make_agg(vpad, d_sup, per_w):
    n_chunks = per_w // CH
    rows_per_tile = vpad // NS
    mesh = plsc.VectorSubcoreMesh(core_axis_name="c", subcore_axis_name="s")

    @functools.partial(
        pl.kernel,
        mesh=mesh,
        out_type=jax.ShapeDtypeStruct((NC, vpad, d_sup), jnp.float32),
        scratch_types=[
            pltpu.VMEM((CH,), jnp.int32),
            pltpu.VMEM((CH,), jnp.int32),
            pltpu.VMEM((CH, d_sup), jnp.float32),
            pltpu.VMEM_SHARED((vpad, d_sup), jnp.float32),
            pltpu.SemaphoreType.DMA,
        ],
        compiler_params=pltpu.CompilerParams(use_tc_tiling_on_sc=False),
    )
    def agg(w1_hbm, dst_hbm, src_hbm, zeros_hbm, out_hbm, dsti, srci, rows, acc, sem):
        c = lax.axis_index("c")
        s = lax.axis_index("s")
        wid = s * NC + c
        base_w = wid * per_w

        r0 = s * rows_per_tile
        pltpu.sync_copy(zeros_hbm.at[pl.ds(r0, rows_per_tile)],
                        acc.at[pl.ds(r0, rows_per_tile)])
        plsc.subcore_barrier()

        def chunk(k, _):
            off = base_w + k * CH
            pltpu.sync_copy(dst_hbm.at[pl.ds(off, CH)], dsti)
            pltpu.sync_copy(src_hbm.at[pl.ds(off, CH)], srci)
            pltpu.async_copy(w1_hbm.at[srci], rows, sem).wait()
            pltpu.sync_copy(rows, acc.at[dsti], add=True)
            return _

        lax.fori_loop(0, n_chunks, chunk, None)
        plsc.subcore_barrier()
        pltpu.sync_copy(acc.at[pl.ds(r0, rows_per_tile)],
                        out_hbm.at[c, pl.ds(r0, rows_per_tile)])

    return agg


@jax.jit
def kernel(verts, edges, w0_w, w0_b, w1_w, w1_b):
    v, d_in = verts.shape
    d_out = w0_w.shape[0]
    d_sup = w1_w.shape[0]
    e = edges.shape[0]
    vb = 1000
    grid = v // vb

    vw0, vw1 = pl.pallas_call(
        _proj_body,
        grid=(grid,),
        in_specs=[
            pl.BlockSpec((vb, d_in), lambda i: (i, 0)),
            pl.BlockSpec((d_in, d_out), lambda i: (0, 0)),
            pl.BlockSpec((d_in, d_sup), lambda i: (0, 0)),
            pl.BlockSpec((1, d_out), lambda i: (0, 0)),
            pl.BlockSpec((1, d_sup), lambda i: (0, 0)),
        ],
        out_specs=[
            pl.BlockSpec((vb, d_out), lambda i: (i, 0)),
            pl.BlockSpec((vb, d_sup), lambda i: (i, 0)),
        ],
        out_shape=[
            jax.ShapeDtypeStruct((v, d_out), jnp.float32),
            jax.ShapeDtypeStruct((v, d_sup), jnp.float32),
        ],
    )(verts, w0_w.T, w1_w.T, w0_b[None, :], w1_b[None, :])

    dst = jnp.concatenate([edges[:, 0], edges[:, 1]])
    src = jnp.concatenate([edges[:, 1], edges[:, 0]])
    per_w = -(-2 * e // (NW * CH)) * CH
    total = per_w * NW
    pad_n = total - 2 * e
    vpad = -(-(v + 1) // (NS * 8)) * (NS * 8)
    dst = jnp.concatenate([dst, jnp.full((pad_n,), v, jnp.int32)])
    src = jnp.concatenate([src, jnp.zeros((pad_n,), jnp.int32)])
    zeros = jnp.zeros((vpad, d_sup), jnp.float32)

    partials = _make_agg(vpad, d_sup, per_w)(vw1, dst, src, zeros)

    out = pl.pallas_call(
        _combine_body,
        grid=(grid,),
        in_specs=[
            pl.BlockSpec((vb, d_out), lambda i: (i, 0)),
            pl.BlockSpec((vb, d_sup), lambda i: (i, 0)),
            pl.BlockSpec((vb, d_sup), lambda i: (i, 0)),
        ],
        out_specs=pl.BlockSpec((vb, d_out), lambda i: (i, 0)),
        out_shape=jax.ShapeDtypeStruct((v, d_out), jnp.float32),
    )(vw0, partials[0, :v], partials[1, :v])
    return out


# speedup vs baseline: 1.1288x; 1.1288x over previous
import functools

import jax
import jax.numpy as jnp
from jax import lax
from jax.experimental import pallas as pl
from jax.experimental.pallas import tpu as pltpu
from jax.experimental.pallas import tpu_sc as plsc

NC = 2
NS = 16
NW = NC * NS
CH = 128
NB = 4


def _proj_body(x_ref, w0t_ref, w1t_ref, b0_ref, b1_ref, o0_ref, o1_ref):
    x = x_ref[...]
    o0_ref[...] = jnp.dot(x, w0t_ref[...], preferred_element_type=jnp.float32) + b0_ref[...]
    o1_ref[...] = jnp.dot(x, w1t_ref[...], preferred_element_type=jnp.float32) + b1_ref[...]


def _combine_body(vw0_ref, p0_ref, p1_ref, o_ref):
    s = p0_ref[...] + p1_ref[...]
    o_ref[...] = vw0_ref[...] + jnp.concatenate([s, jnp.zeros_like(s)], axis=1)


def _make_agg(vpad, d_sup, per_w):
    n_chunks = per_w // CH
    n_rounds = n_chunks // NB
    rows_per_tile = vpad // NS
    mesh = plsc.VectorSubcoreMesh(core_axis_name="c", subcore_axis_name="s")

    @functools.partial(
        pl.kernel,
        mesh=mesh,
        out_type=jax.ShapeDtypeStruct((NC, vpad, d_sup), jnp.float32),
        scratch_types=[
            pltpu.VMEM((n_chunks, CH), jnp.int32),
            pltpu.VMEM((n_chunks, CH), jnp.int32),
            [pltpu.VMEM((CH, d_sup), jnp.float32) for _ in range(NB)],
            pltpu.VMEM_SHARED((vpad, d_sup), jnp.float32),
            [pltpu.SemaphoreType.DMA for _ in range(NB)],
        ],
        compiler_params=pltpu.CompilerParams(use_tc_tiling_on_sc=False),
    )
    def agg(w1_hbm, dst_hbm, src_hbm, zeros_hbm, out_hbm, dsti, srci, rows, acc, sems):
        c = lax.axis_index("c")
        s = lax.axis_index("s")
        wid = s * NC + c

        gd = pltpu.async_copy(dst_hbm.at[wid], dsti, sems[0])
        gs = pltpu.async_copy(src_hbm.at[wid], srci, sems[1])
        r0 = s * rows_per_tile
        pltpu.sync_copy(zeros_hbm.at[pl.ds(r0, rows_per_tile)],
                        acc.at[pl.ds(r0, rows_per_tile)])
        gd.wait()
        gs.wait()
        plsc.subcore_barrier()

        for b in range(NB):
            pltpu.async_copy(w1_hbm.at[srci.at[b]], rows[b], sems[b])

        def round_body(r, _):
            k0 = r * NB
            for b in range(NB):
                k = k0 + b
                pltpu.make_async_copy(w1_hbm.at[srci.at[k]], rows[b], sems[b]).wait()
                pltpu.sync_copy(rows[b], acc.at[dsti.at[k]], add=True)

                @pl.when(k + NB < n_chunks)
                def _():
                    pltpu.async_copy(w1_hbm.at[srci.at[k + NB]], rows[b], sems[b])
            return _

        lax.fori_loop(0, n_rounds, round_body, None)
        plsc.subcore_barrier()
        pltpu.sync_copy(acc.at[pl.ds(r0, rows_per_tile)],
                        out_hbm.at[c, pl.ds(r0, rows_per_tile)])

    return agg


@jax.jit
def kernel(verts, edges, w0_w, w0_b, w1_w, w1_b):
    v, d_in = verts.shape
    d_out = w0_w.shape[0]
    d_sup = w1_w.shape[0]
    e = edges.shape[0]
    vb = 1000
    grid = v // vb

    vw0, vw1 = pl.pallas_call(
        _proj_body,
        grid=(grid,),
        in_specs=[
            pl.BlockSpec((vb, d_in), lambda i: (i, 0)),
            pl.BlockSpec((d_in, d_out), lambda i: (0, 0)),
            pl.BlockSpec((d_in, d_sup), lambda i: (0, 0)),
            pl.BlockSpec((1, d_out), lambda i: (0, 0)),
            pl.BlockSpec((1, d_sup), lambda i: (0, 0)),
        ],
        out_specs=[
            pl.BlockSpec((vb, d_out), lambda i: (i, 0)),
            pl.BlockSpec((vb, d_sup), lambda i: (i, 0)),
        ],
        out_shape=[
            jax.ShapeDtypeStruct((v, d_out), jnp.float32),
            jax.ShapeDtypeStruct((v, d_sup), jnp.float32),
        ],
    )(verts, w0_w.T, w1_w.T, w0_b[None, :], w1_b[None, :])

    dst = jnp.concatenate([edges[:, 0], edges[:, 1]])
    src = jnp.concatenate([edges[:, 1], edges[:, 0]])
    per_w = -(-2 * e // (NW * CH * NB)) * (CH * NB)
    total = per_w * NW
    pad_n = total - 2 * e
    vpad = -(-(v + 1) // (NS * 8)) * (NS * 8)
    dst = jnp.concatenate([dst, jnp.full((pad_n,), v, jnp.int32)])
    src = jnp.concatenate([src, jnp.zeros((pad_n,), jnp.int32)])
    dst = dst.reshape(NW, per_w // CH, CH)
    src = src.reshape(NW, per_w // CH, CH)
    zeros = jnp.zeros((vpad, d_sup), jnp.float32)

    partials = _make_agg(vpad, d_sup, per_w)(vw1, dst, src, zeros)

    out = pl.pallas_call(
        _combine_body,
        grid=(grid,),
        in_specs=[
            pl.BlockSpec((vb, d_out), lambda i: (i, 0)),
            pl.BlockSpec((vb, d_sup), lambda i: (i, 0)),
            pl.BlockSpec((vb, d_sup), lambda i: (i, 0)),
        ],
        out_specs=pl.BlockSpec((vb, d_out), lambda i: (i, 0)),
        out_shape=jax.ShapeDtypeStruct((v, d_out), jnp.float32),
    )(vw0, partials[0, :v], partials[1, :v])
    return out


# speedup vs baseline: 2.9037x; 2.5724x over previous
import functools

import jax
import jax.numpy as jnp
from jax import lax
from jax.experimental import pallas as pl
from jax.experimental.pallas import tpu as pltpu
from jax.experimental.pallas import tpu_sc as plsc

NC = 2
NS = 16
CH = 128
NB = 2


def _proj_body(x_ref, w0t_ref, w1t_ref, b0_ref, b1_ref, o0_ref, o1_ref):
    x = x_ref[...]
    o0_ref[...] = jnp.dot(x, w0t_ref[...], preferred_element_type=jnp.float32) + b0_ref[...]
    o1_ref[...] = jnp.dot(x, w1t_ref[...], preferred_element_type=jnp.float32) + b1_ref[...]


def _combine_body(vw0_ref, p0_ref, p1_ref, o_ref):
    ns = jnp.concatenate([p0_ref[...], p1_ref[...]], axis=1)
    o_ref[...] = vw0_ref[...] + jnp.concatenate([ns, jnp.zeros_like(ns)], axis=1)


def _make_agg(v, vpad, hc, per_w):
    n_chunks = per_w // CH
    n_rounds = n_chunks // NB
    rows_per_tile = vpad // NS
    stage_per_tile = v // NS
    mesh = plsc.VectorSubcoreMesh(core_axis_name="c", subcore_axis_name="s")

    @functools.partial(
        pl.kernel,
        mesh=mesh,
        out_type=jax.ShapeDtypeStruct((NC, vpad, hc), jnp.float32),
        scratch_types=[
            pltpu.VMEM((n_chunks, CH), jnp.int32),
            pltpu.VMEM((n_chunks, CH), jnp.int32),
            [pltpu.VMEM((CH, hc), jnp.float32) for _ in range(NB)],
            pltpu.VMEM((8, hc), jnp.float32),
            pltpu.VMEM_SHARED((vpad, hc), jnp.float32),
            pltpu.VMEM_SHARED((vpad, hc), jnp.float32),
            [pltpu.SemaphoreType.DMA for _ in range(NB)],
        ],
        compiler_params=pltpu.CompilerParams(use_tc_tiling_on_sc=False),
    )
    def agg(w1c_hbm, dst_hbm, src_hbm, out_hbm, dsti, srci, rows, zbuf, acc, w1s, sems):
        c = lax.axis_index("c")
        s = lax.axis_index("s")

        gd = pltpu.async_copy(dst_hbm.at[s], dsti, sems[0])
        gs = pltpu.async_copy(src_hbm.at[s], srci, sems[1])
        r0 = s * rows_per_tile
        sv = s * stage_per_tile
        pltpu.sync_copy(w1c_hbm.at[c, pl.ds(sv, stage_per_tile)],
                        w1s.at[pl.ds(sv, stage_per_tile)])
        for i in range(8):
            for j in range(hc // 16):
                zbuf[i, pl.ds(16 * j, 16)] = jnp.zeros((16,), jnp.float32)

        def zero_body(t, carry):
            pltpu.sync_copy(zbuf, acc.at[pl.ds(r0 + t * 8, 8)])
            return carry

        lax.fori_loop(0, rows_per_tile // 8, zero_body, None)
        gd.wait()
        gs.wait()
        plsc.subcore_barrier()

        for b in range(NB):
            pltpu.async_copy(w1s.at[srci.at[b]], rows[b], sems[b])

        def round_body(r, carry):
            k0 = r * NB
            for b in range(NB):
                k = k0 + b
                pltpu.make_async_copy(w1s.at[srci.at[k]], rows[b], sems[b]).wait()
                pltpu.sync_copy(rows[b], acc.at[dsti.at[k]], add=True)

                @pl.when(k + NB < n_chunks)
                def _():
                    pltpu.async_copy(w1s.at[srci.at[k + NB]], rows[b], sems[b])
            return carry

        lax.fori_loop(0, n_rounds, round_body, None)
        plsc.subcore_barrier()
        pltpu.sync_copy(acc.at[pl.ds(r0, rows_per_tile)],
                        out_hbm.at[c, pl.ds(r0, rows_per_tile)])

    return agg


@jax.jit
def kernel(verts, edges, w0_w, w0_b, w1_w, w1_b):
    v, d_in = verts.shape
    d_out = w0_w.shape[0]
    d_sup = w1_w.shape[0]
    hc = d_sup // NC
    e = edges.shape[0]
    vb = 1000
    grid = v // vb

    vw0, vw1 = pl.pallas_call(
        _proj_body,
        grid=(grid,),
        in_specs=[
            pl.BlockSpec((vb, d_in), lambda i: (i, 0)),
            pl.BlockSpec((d_in, d_out), lambda i: (0, 0)),
            pl.BlockSpec((d_in, d_sup), lambda i: (0, 0)),
            pl.BlockSpec((1, d_out), lambda i: (0, 0)),
            pl.BlockSpec((1, d_sup), lambda i: (0, 0)),
        ],
        out_specs=[
            pl.BlockSpec((vb, d_out), lambda i: (i, 0)),
            pl.BlockSpec((vb, d_sup), lambda i: (i, 0)),
        ],
        out_shape=[
            jax.ShapeDtypeStruct((v, d_out), jnp.float32),
            jax.ShapeDtypeStruct((v, d_sup), jnp.float32),
        ],
    )(verts, w0_w.T, w1_w.T, w0_b[None, :], w1_b[None, :])

    dst = jnp.concatenate([edges[:, 0], edges[:, 1]])
    src = jnp.concatenate([edges[:, 1], edges[:, 0]])
    per_w = -(-2 * e // (NS * CH * NB)) * (CH * NB)
    pad_n = per_w * NS - 2 * e
    vpad = -(-(v + 1) // (NS * 8)) * (NS * 8)
    dst = jnp.concatenate([dst, jnp.full((pad_n,), v, jnp.int32)])
    src = jnp.concatenate([src, jnp.zeros((pad_n,), jnp.int32)])
    dst = dst.reshape(NS, per_w // CH, CH)
    src = src.reshape(NS, per_w // CH, CH)
    w1cols = jnp.stack([vw1[:, :hc], vw1[:, hc:]])

    partials = _make_agg(v, vpad, hc, per_w)(w1cols, dst, src)

    out = pl.pallas_call(
        _combine_body,
        grid=(grid,),
        in_specs=[
            pl.BlockSpec((vb, d_out), lambda i: (i, 0)),
            pl.BlockSpec((vb, hc), lambda i: (i, 0)),
            pl.BlockSpec((vb, hc), lambda i: (i, 0)),
        ],
        out_specs=pl.BlockSpec((vb, d_out), lambda i: (i, 0)),
        out_shape=jax.ShapeDtypeStruct((v, d_out), jnp.float32),
    )(vw0, partials[0, :v], partials[1, :v])
    return out


# speedup vs baseline: 3.0142x; 1.0380x over previous
import functools

import jax
import jax.numpy as jnp
from jax import lax
from jax.experimental import pallas as pl
from jax.experimental.pallas import tpu as pltpu
from jax.experimental.pallas import tpu_sc as plsc

NC = 2
NS = 16
CH = 128
NB = 2


def _proj_body(x_ref, w0t_ref, w1ta_ref, w1tb_ref, b0_ref, b1a_ref, b1b_ref,
               o0_ref, o1_ref):
    x = x_ref[...]
    o0_ref[...] = jnp.dot(x, w0t_ref[...], preferred_element_type=jnp.float32) + b0_ref[...]
    o1_ref[0] = jnp.dot(x, w1ta_ref[...], preferred_element_type=jnp.float32) + b1a_ref[...]
    o1_ref[1] = jnp.dot(x, w1tb_ref[...], preferred_element_type=jnp.float32) + b1b_ref[...]


def _combine_body(vw0_ref, p0_ref, p1_ref, o_ref):
    ns = jnp.concatenate([p0_ref[0], p1_ref[0]], axis=1)
    o_ref[...] = vw0_ref[...] + jnp.concatenate([ns, jnp.zeros_like(ns)], axis=1)


def _make_agg(v, vpad, hc, per_w):
    n_chunks = per_w // CH
    n_rounds = n_chunks // NB
    rows_per_tile = vpad // NS
    stage_per_tile = v // NS
    mesh = plsc.VectorSubcoreMesh(core_axis_name="c", subcore_axis_name="s")

    @functools.partial(
        pl.kernel,
        mesh=mesh,
        out_type=jax.ShapeDtypeStruct((NC, vpad, hc), jnp.float32),
        scratch_types=[
            pltpu.VMEM((n_chunks, CH), jnp.int32),
            [pltpu.VMEM((CH,), jnp.int32) for _ in range(NB)],
            [pltpu.VMEM((CH,), jnp.int32) for _ in range(NB)],
            [pltpu.VMEM((CH, hc), jnp.float32) for _ in range(NB)],
            [pltpu.VMEM((CH, hc), jnp.float32) for _ in range(NB)],
            pltpu.VMEM((rows_per_tile, hc), jnp.float32),
            pltpu.VMEM_SHARED((vpad, hc), jnp.float32),
            pltpu.VMEM_SHARED((vpad, hc), jnp.float32),
            [pltpu.SemaphoreType.DMA for _ in range(2 * NB)],
        ],
        compiler_params=pltpu.CompilerParams(use_tc_tiling_on_sc=False),
    )
    def agg(w1c_hbm, epk_hbm, out_hbm, epki, dsti, srci, rowsa, rowsb, zbuf,
            acc, w1s, sems):
        c = lax.axis_index("c")
        s = lax.axis_index("s")

        ge = pltpu.async_copy(epk_hbm.at[s], epki, sems[0])
        r0 = s * rows_per_tile
        sv = s * stage_per_tile
        pltpu.sync_copy(w1c_hbm.at[c, pl.ds(sv, stage_per_tile)],
                        w1s.at[pl.ds(sv, stage_per_tile)])

        def zstore(i, carry):
            for j in range(hc // 16):
                zbuf[i, pl.ds(16 * j, 16)] = jnp.zeros((16,), jnp.float32)
            return carry

        lax.fori_loop(0, rows_per_tile, zstore, None)
        pltpu.sync_copy(zbuf, acc.at[pl.ds(r0, rows_per_tile)])
        ge.wait()
        plsc.subcore_barrier()

        def unpack(k, b):
            for j in range(CH // 16):
                w = epki[k, pl.ds(16 * j, 16)]
                dsti[b][pl.ds(16 * j, 16)] = w >> 16
                srci[b][pl.ds(16 * j, 16)] = w & 0xFFFF

        def fire(k, b):
            pltpu.async_copy(w1s.at[srci[b]], rowsa[b], sems[2 * b])
            pltpu.async_copy(w1s.at[dsti[b]], rowsb[b], sems[2 * b + 1])

        for b in range(NB):
            unpack(b, b)
            fire(b, b)

        def round_body(r, carry):
            k0 = r * NB
            for b in range(NB):
                k = k0 + b
                pltpu.make_async_copy(w1s.at[srci[b]], rowsa[b], sems[2 * b]).wait()
                pltpu.sync_copy(rowsa[b], acc.at[dsti[b]], add=True)
                pltpu.make_async_copy(w1s.at[dsti[b]], rowsb[b], sems[2 * b + 1]).wait()
                pltpu.sync_copy(rowsb[b], acc.at[srci[b]], add=True)

                @pl.when(k + NB < n_chunks)
                def _():
                    unpack(k + NB, b)
                    fire(k + NB, b)
            return carry

        lax.fori_loop(0, n_rounds, round_body, None)
        plsc.subcore_barrier()
        pltpu.sync_copy(acc.at[pl.ds(r0, rows_per_tile)],
                        out_hbm.at[c, pl.ds(r0, rows_per_tile)])

    return agg


@jax.jit
def kernel(verts, edges, w0_w, w0_b, w1_w, w1_b):
    v, d_in = verts.shape
    d_out = w0_w.shape[0]
    d_sup = w1_w.shape[0]
    hc = d_sup // NC
    e = edges.shape[0]
    vb = 1000
    grid = v // vb
    vpad = -(-(v + 1) // (NS * 8)) * (NS * 8)

    w1t = w1_w.T

    vw0, vw1c = pl.pallas_call(
        _proj_body,
        grid=(grid,),
        in_specs=[
            pl.BlockSpec((vb, d_in), lambda i: (i, 0)),
            pl.BlockSpec((d_in, d_out), lambda i: (0, 0)),
            pl.BlockSpec((d_in, hc), lambda i: (0, 0)),
            pl.BlockSpec((d_in, hc), lambda i: (0, 0)),
            pl.BlockSpec((1, d_out), lambda i: (0, 0)),
            pl.BlockSpec((1, hc), lambda i: (0, 0)),
            pl.BlockSpec((1, hc), lambda i: (0, 0)),
        ],
        out_specs=[
            pl.BlockSpec((vb, d_out), lambda i: (i, 0)),
            pl.BlockSpec((NC, vb, hc), lambda i: (0, i, 0)),
        ],
        out_shape=[
            jax.ShapeDtypeStruct((v, d_out), jnp.float32),
            jax.ShapeDtypeStruct((NC, v, hc), jnp.float32),
        ],
    )(verts, w0_w.T, w1t[:, :hc], w1t[:, hc:], w0_b[None, :],
      w1_b[None, :hc], w1_b[None, hc:])

    epk = jnp.left_shift(edges[:, 0], 16) | edges[:, 1]
    per_w = -(-e // (NS * CH * NB)) * (CH * NB)
    pad_n = per_w * NS - e
    epk = jnp.concatenate([epk, jnp.full((pad_n,), (v << 16) | v, jnp.int32)])
    epk = epk.reshape(NS, per_w // CH, CH)

    partials = _make_agg(v, vpad, hc, per_w)(vw1c, epk)

    out = pl.pallas_call(
        _combine_body,
        grid=(grid,),
        in_specs=[
            pl.BlockSpec((vb, d_out), lambda i: (i, 0)),
            pl.BlockSpec((1, vb, hc), lambda i: (0, i, 0)),
            pl.BlockSpec((1, vb, hc), lambda i: (1, i, 0)),
        ],
        out_specs=pl.BlockSpec((vb, d_out), lambda i: (i, 0)),
        out_shape=jax.ShapeDtypeStruct((v, d_out), jnp.float32),
    )(vw0, partials, partials)
    return out
